# baseline (speedup 1.0000x reference)
import functools
import math

import jax
import jax.numpy as jnp
import numpy as np
from jax.experimental import pallas as pl

NUM_LEVELS = 3
BASE_SCALE = 0.25
RESOLUTION = 3
TRUNCATE_K = 128
KNN = 32


def _get_base():
    num = RESOLUTION ** 3
    base = np.zeros((num, 3), np.float32)
    for i in range(num):
        q = i
        for j in range(3):
            base[i][2 - j] = q % RESOLUTION - 1
            q = q // RESOLUTION
    return jnp.asarray(base)


def _conv1d(x, W, b):
    return jnp.einsum('oi,bin->bon', W, x) + b[None, :, None]


def _conv2d(x, W, b):
    return jnp.einsum('oi,bihw->bohw', W, x) + b[None, :, None, None]


def _group_norm(x, gamma, beta, g, eps=1e-5):
    b, C = x.shape[0], x.shape[1]
    rest = x.shape[2:]
    xr = x.reshape(b, g, -1)
    mean = xr.mean(axis=2, keepdims=True)
    var = xr.var(axis=2, keepdims=True)
    xn = ((xr - mean) / jnp.sqrt(var + eps)).reshape(x.shape)
    sh = (1, C) + (1,) * len(rest)
    return xn * gamma.reshape(sh) + beta.reshape(sh)


def _prelu(x, a):
    return jnp.where(x >= 0, x, a * x)


def _scatter_add_last(vals, idx, nbins):
    b, n, k = vals.shape
    bi = jnp.arange(b)[:, None, None]
    ni = jnp.arange(n)[None, :, None]
    return jnp.zeros((b, n, nbins), vals.dtype).at[bi, ni, idx].add(vals)


def _scatter_max_last(vals, idx, nbins):
    mask = idx[..., None] == jnp.arange(nbins)[None, None, None, :]
    masked = jnp.where(mask, vals[..., None], -jnp.inf)
    return masked.max(axis=2), masked.argmax(axis=2)


def _corr_matmul_body(f1_ref, f2_ref, o_ref, *, scale):
    f1 = f1_ref[0]
    f2 = f2_ref[0]
    o_ref[0] = jax.lax.dot_general(
        f1, f2, (((0,), (0,)), ((), ())),
        preferred_element_type=jnp.float32) * scale


def _corr_pallas(fmap1, fmap2):
    b, d, n = fmap1.shape
    BN = 512
    scale = 1.0 / math.sqrt(float(d))
    return pl.pallas_call(
        functools.partial(_corr_matmul_body, scale=scale),
        grid=(b, n // BN),
        in_specs=[
            pl.BlockSpec((1, d, BN), lambda i, j: (i, 0, j)),
            pl.BlockSpec((1, d, n), lambda i, j: (i, 0, 0)),
        ],
        out_specs=pl.BlockSpec((1, BN, n), lambda i, j: (i, j, 0)),
        out_shape=jax.ShapeDtypeStruct((b, n, n), jnp.float32),
    )(fmap1, fmap2)


def kernel(fmap1, fmap2, xyz2, coords, W_out1, b_out1, g_out, be_out, a_out,
           W_out2, b_out2, W_v1, b_v1, g_v, be_v, a_v, W_v2, b_v2,
           W_k1, b_k1, g_k, be_k, a_k, W_ko, b_ko):
    b, dim, n_p = fmap1.shape
    K = TRUNCATE_K
    num = RESOLUTION ** 3
    corr = _corr_pallas(fmap1, fmap2)
    truncated_corr, indx = jax.lax.top_k(corr, K)
    truncate_xyz2 = jax.vmap(lambda p, i: p[i])(xyz2, indx)
    diff = truncate_xyz2 - coords[:, :, None, :]
    base = _get_base()
    corr_feature = []
    for i in range(NUM_LEVELS):
        r = BASE_SCALE * 2 ** i
        dv = jnp.round(diff / r)
        valid = jnp.all(jnp.abs(dv) <= np.floor(RESOLUTION / 2), axis=-1)
        dv = dv + 1
        cube = dv[..., 0] * RESOLUTION ** 2 + dv[..., 1] * RESOLUTION + dv[..., 2]
        cube = jnp.where(valid, cube, 0.0).astype(jnp.int32)
        vf = valid.astype(jnp.float32)
        ca = _scatter_add_last(truncated_corr * vf, cube, num)
        cc = jnp.clip(_scatter_add_last(vf, cube, num), 1.0, float(n_p))
        corr_feature.append(jnp.transpose(ca / cc, (0, 2, 1)))
    r8 = BASE_SCALE * 2 ** 3
    R = 2 * r8
    dv = jnp.round(diff / R)
    validc = jnp.all(jnp.abs(dv) <= 1.5, axis=-1)
    dvi = dv + 1
    cidx = dvi[..., 0] * RESOLUTION ** 2 + dvi[..., 1] * RESOLUTION + dvi[..., 2]
    cidx_i = jnp.where(validc, cidx, 0.0).astype(jnp.int32)
    full = jnp.where(validc, cidx_i + 1, 0)
    mvals = truncated_corr * validc.astype(jnp.float32)
    center_corr, center_idx = _scatter_max_last(mvals, full, num + 1)
    center_idx = center_idx[:, :, 1:]
    cc_t = center_corr[:, :, 1:]
    mask_oob = (center_idx >= K) | (center_idx < 0) | (cc_t <= 0)
    center_idx = jnp.where(mask_oob, 0, center_idx)
    center_coord = jnp.take_along_axis(
        truncate_xyz2,
        jnp.broadcast_to(center_idx[..., None], center_idx.shape + (3,)), axis=2)
    idx_scatter = jnp.full((b, n_p, K), num, jnp.int32)
    moves = []
    for k in range(num):
        cck = center_coord[:, :, k:k + 1, :]
        vkc = (coords + base[k][None, None, :] * R)[:, :, None, :]
        ckm = jnp.clip(cck - vkc, -(R - r8) / 2, (R - r8) / 2) + vkc
        moves.append(ckm)
        dis_center = (truncate_xyz2 - ckm) / r8
        replace = jnp.all(jnp.abs(dis_center) <= 0.5, axis=-1) & (~mask_oob[:, :, k:k + 1])
        idx_scatter = jnp.where(replace, jnp.int32(k), idx_scatter)
    voxel_xyz = jnp.concatenate(moves, axis=2)
    ca = _scatter_add_last(truncated_corr, idx_scatter, num + 1)
    cnt = _scatter_add_last(jnp.ones_like(truncated_corr), idx_scatter, num + 1)
    voxel_corr = ca[:, :, :num] / jnp.clip(cnt[:, :, :num], 1.0, float(n_p))
    vin = jnp.concatenate(
        [voxel_corr[:, None], jnp.transpose(voxel_xyz, (0, 3, 1, 2))], axis=1)
    vfe = _conv2d(vin, W_v1, b_v1)
    vfe = _prelu(_group_norm(vfe, g_v, be_v, 8), a_v)
    vfe = _conv2d(vfe, W_v2, b_v2)
    corr_feature.append(jnp.transpose(vfe[:, 0], (0, 2, 1)))
    feat = jnp.concatenate(corr_feature, axis=1)
    h = _conv1d(feat, W_out1, b_out1)
    h = _prelu(_group_norm(h, g_out, be_out, 8), a_out)
    voxel_out = _conv1d(h, W_out2, b_out2)
    dist = jnp.sum(diff ** 2, axis=-1)
    _, neighbors = jax.lax.top_k(-dist, KNN)
    knn_corr = jnp.take_along_axis(truncated_corr, neighbors, axis=2)[:, None]
    knn_xyz = jnp.take_along_axis(
        truncate_xyz2,
        jnp.broadcast_to(neighbors[..., None], neighbors.shape + (3,)), axis=2)
    knn_xyz = jnp.transpose(knn_xyz, (0, 3, 1, 2)) - jnp.transpose(coords, (0, 2, 1))[..., None]
    kin = jnp.concatenate([knn_corr, knn_xyz], axis=1)
    kf = _conv2d(kin, W_k1, b_k1)
    kf = _prelu(_group_norm(kf, g_k, be_k, 8), a_k)
    kf = jnp.max(kf, axis=3)
    knn_feat = _conv1d(kf, W_ko, b_ko)
    return voxel_out + knn_feat


# speedup vs baseline: 2.0666x; 2.0666x over previous
import functools
import math

import jax
import jax.numpy as jnp
import numpy as np
from jax.experimental import pallas as pl

NUM_LEVELS = 3
BASE_SCALE = 0.25
RESOLUTION = 3
TRUNCATE_K = 128
KNN = 32


def _get_base():
    num = RESOLUTION ** 3
    base = np.zeros((num, 3), np.float32)
    for i in range(num):
        q = i
        for j in range(3):
            base[i][2 - j] = q % RESOLUTION - 1
            q = q // RESOLUTION
    return jnp.asarray(base)


def _conv1d(x, W, b):
    return jnp.einsum('oi,bin->bon', W, x) + b[None, :, None]


def _conv2d(x, W, b):
    return jnp.einsum('oi,bihw->bohw', W, x) + b[None, :, None, None]


def _group_norm(x, gamma, beta, g, eps=1e-5):
    b, C = x.shape[0], x.shape[1]
    rest = x.shape[2:]
    xr = x.reshape(b, g, -1)
    mean = xr.mean(axis=2, keepdims=True)
    var = xr.var(axis=2, keepdims=True)
    xn = ((xr - mean) / jnp.sqrt(var + eps)).reshape(x.shape)
    sh = (1, C) + (1,) * len(rest)
    return xn * gamma.reshape(sh) + beta.reshape(sh)


def _prelu(x, a):
    return jnp.where(x >= 0, x, a * x)


def _scatter_add_last(vals, idx, nbins):
    b, n, k = vals.shape
    bi = jnp.arange(b)[:, None, None]
    ni = jnp.arange(n)[None, :, None]
    return jnp.zeros((b, n, nbins), vals.dtype).at[bi, ni, idx].add(vals)


def _scatter_max_last(vals, idx, nbins):
    mask = idx[..., None] == jnp.arange(nbins)[None, None, None, :]
    masked = jnp.where(mask, vals[..., None], -jnp.inf)
    return masked.max(axis=2), masked.argmax(axis=2)


_MIN32 = -2147483648


def _corr_topk_body(f1_ref, f2_ref, xyz_ref,
                    tc_ref, dx_ref, dy_ref, dz_ref, *, bn, n, k):
    i32, f32 = jnp.int32, jnp.float32
    nck = n // 128
    f1 = f1_ref[0]
    f2 = f2_ref[0]
    corr = jax.lax.dot_general(
        f1, f2, (((0,), (0,)), ((), ())), preferred_element_type=f32)

    corr3 = corr.reshape(bn, nck, 128)
    kbits = jax.lax.bitcast_convert_type(corr3, i32)
    ks3 = kbits ^ ((kbits >> 31) & jnp.int32(0x7FFFFFFF))

    kf = jnp.float32(k)

    def _bisect(i, tb):
        bit = jnp.left_shift(jnp.int32(1), 31 - i)
        cand = tb | bit
        cand_s = (cand ^ jnp.int32(_MIN32))[:, :, None]
        cnt = jnp.sum(jnp.sum((ks3 >= cand_s).astype(f32), axis=2),
                      axis=1, keepdims=True)
        return jnp.where(cnt >= kf, cand, tb)

    tb = jax.lax.fori_loop(0, 32, _bisect, jnp.zeros((bn, 1), i32))
    ts3 = (tb ^ jnp.int32(_MIN32))[:, :, None]

    gt3 = ks3 > ts3
    eq3b = ks3 == ts3
    cnt_gt = jnp.sum(jnp.sum(gt3.astype(f32), axis=2), axis=1, keepdims=True)
    m_tie = kf - cnt_gt

    ir = jax.lax.broadcasted_iota(i32, (128, 128), 0)
    ic = jax.lax.broadcasted_iota(i32, (128, 128), 1)
    tri = (ir <= ic).astype(f32)
    ir2 = jax.lax.broadcasted_iota(i32, (nck, nck), 0)
    ic2 = jax.lax.broadcasted_iota(i32, (nck, nck), 1)
    tri_c_incl = (ir2 <= ic2).astype(f32)

    eq3 = eq3b.astype(f32)
    wc_eq = jax.lax.dot_general(eq3, tri, (((2,), (0,)), ((), ())),
                                preferred_element_type=f32)
    tot_eq = jnp.sum(eq3, axis=2)
    oi_eq = jax.lax.dot_general(tot_eq, tri_c_incl, (((1,), (0,)), ((), ())),
                                preferred_element_type=f32)
    off_eq = oi_eq - tot_eq
    rank_excl = off_eq[:, :, None] + (wc_eq - eq3)
    sel = gt3 | (eq3b & (rank_excl < m_tie[:, :, None]))

    sf = sel.astype(f32)
    cw = jax.lax.dot_general(sf, tri, (((2,), (0,)), ((), ())),
                             preferred_element_type=f32)
    m_c = jnp.sum(sf, axis=2)
    oi = jax.lax.dot_general(m_c, tri_c_incl, (((1,), (0,)), ((), ())),
                             preferred_element_type=f32)
    se = oi - m_c

    kio_f = jax.lax.broadcasted_iota(i32, (bn, k), 1).astype(f32)
    cstar = jnp.zeros((bn, k), i32)
    for c in range(nck):
        cstar = cstar + (oi[:, c:c + 1] <= kio_f).astype(i32)
    se_k = jnp.take_along_axis(se, cstar, axis=1)
    s_loc = kio_f - se_k

    cwi = cw.astype(i32)
    sio = jax.lax.broadcasted_iota(i32, (bn, nck, 128), 2)

    def _ssorted(i, q3):
        b_ = jnp.int32(64) >> i
        cand = q3 + b_
        cc = jnp.take_along_axis(cwi, cand - 1, axis=2)
        return jnp.where(cc <= sio, cand, q3)

    q3 = jax.lax.fori_loop(0, 7, _ssorted, jnp.zeros((bn, nck, 128), i32))

    s_loc_i = s_loc.astype(i32)
    out_v = jnp.zeros((bn, k), f32)
    out_x = jnp.zeros((bn, k), f32)
    out_y = jnp.zeros((bn, k), f32)
    out_z = jnp.zeros((bn, k), f32)
    xyz = xyz_ref[0]
    for c in range(nck):
        msk = cstar == c
        qv = jnp.take_along_axis(q3[:, c, :], s_loc_i, axis=1)
        vv = jnp.take_along_axis(corr3[:, c, :], qv, axis=1)
        xc = jnp.broadcast_to(xyz[0:1, c * 128:(c + 1) * 128], (bn, k))
        yc = jnp.broadcast_to(xyz[1:2, c * 128:(c + 1) * 128], (bn, k))
        zc = jnp.broadcast_to(xyz[2:3, c * 128:(c + 1) * 128], (bn, k))
        out_v = jnp.where(msk, vv, out_v)
        out_x = jnp.where(msk, jnp.take_along_axis(xc, qv, axis=1), out_x)
        out_y = jnp.where(msk, jnp.take_along_axis(yc, qv, axis=1), out_y)
        out_z = jnp.where(msk, jnp.take_along_axis(zc, qv, axis=1), out_z)

    am = jnp.argmax(out_v, axis=1, keepdims=True).astype(i32)
    lane = jax.lax.broadcasted_iota(i32, (bn, k), 1)

    def _swap0(p):
        pm = jnp.take_along_axis(p, am, axis=1)
        p0 = p[:, 0:1]
        return jnp.where(lane == 0, pm, jnp.where(lane == am, p0, p))

    out_v, out_x, out_y, out_z = map(_swap0, (out_v, out_x, out_y, out_z))

    scale = jnp.sqrt(jnp.float32(f1.shape[0]))
    tc_ref[0] = out_v / scale
    dx_ref[0] = out_x
    dy_ref[0] = out_y
    dz_ref[0] = out_z


def _corr_topk(fmap1, fmap2, xyz2):
    b, d, n = fmap1.shape
    k = TRUNCATE_K
    BN = 256
    xyz8 = jnp.concatenate(
        [jnp.transpose(xyz2, (0, 2, 1)),
         jnp.zeros((b, 5, n), jnp.float32)], axis=1)
    grid = (b, n // BN)
    out = pl.pallas_call(
        functools.partial(_corr_topk_body, bn=BN, n=n, k=k),
        grid=grid,
        in_specs=[
            pl.BlockSpec((1, d, BN), lambda i, j: (i, 0, j)),
            pl.BlockSpec((1, d, n), lambda i, j: (i, 0, 0)),
            pl.BlockSpec((1, 8, n), lambda i, j: (i, 0, 0)),
        ],
        out_specs=[pl.BlockSpec((1, BN, k), lambda i, j: (i, j, 0))] * 4,
        out_shape=[jax.ShapeDtypeStruct((b, n, k), jnp.float32)] * 4,
    )(fmap1, fmap2, xyz8)
    return out


def kernel(fmap1, fmap2, xyz2, coords, W_out1, b_out1, g_out, be_out, a_out,
           W_out2, b_out2, W_v1, b_v1, g_v, be_v, a_v, W_v2, b_v2,
           W_k1, b_k1, g_k, be_k, a_k, W_ko, b_ko):
    b, dim, n_p = fmap1.shape
    K = TRUNCATE_K
    num = RESOLUTION ** 3
    truncated_corr, ttx, tty, ttz = _corr_topk(fmap1, fmap2, xyz2)
    truncate_xyz2 = jnp.stack([ttx, tty, ttz], axis=-1)
    diff = truncate_xyz2 - coords[:, :, None, :]
    base = _get_base()
    corr_feature = []
    for i in range(NUM_LEVELS):
        r = BASE_SCALE * 2 ** i
        dv = jnp.round(diff / r)
        valid = jnp.all(jnp.abs(dv) <= np.floor(RESOLUTION / 2), axis=-1)
        dv = dv + 1
        cube = dv[..., 0] * RESOLUTION ** 2 + dv[..., 1] * RESOLUTION + dv[..., 2]
        cube = jnp.where(valid, cube, 0.0).astype(jnp.int32)
        vf = valid.astype(jnp.float32)
        ca = _scatter_add_last(truncated_corr * vf, cube, num)
        cc = jnp.clip(_scatter_add_last(vf, cube, num), 1.0, float(n_p))
        corr_feature.append(jnp.transpose(ca / cc, (0, 2, 1)))
    r8 = BASE_SCALE * 2 ** 3
    R = 2 * r8
    dv = jnp.round(diff / R)
    validc = jnp.all(jnp.abs(dv) <= 1.5, axis=-1)
    dvi = dv + 1
    cidx = dvi[..., 0] * RESOLUTION ** 2 + dvi[..., 1] * RESOLUTION + dvi[..., 2]
    cidx_i = jnp.where(validc, cidx, 0.0).astype(jnp.int32)
    full = jnp.where(validc, cidx_i + 1, 0)
    mvals = truncated_corr * validc.astype(jnp.float32)
    center_corr, center_idx = _scatter_max_last(mvals, full, num + 1)
    center_idx = center_idx[:, :, 1:]
    cc_t = center_corr[:, :, 1:]
    mask_oob = (center_idx >= K) | (center_idx < 0) | (cc_t <= 0)
    center_idx = jnp.where(mask_oob, 0, center_idx)
    center_coord = jnp.take_along_axis(
        truncate_xyz2,
        jnp.broadcast_to(center_idx[..., None], center_idx.shape + (3,)), axis=2)
    idx_scatter = jnp.full((b, n_p, K), num, jnp.int32)
    moves = []
    for k in range(num):
        cck = center_coord[:, :, k:k + 1, :]
        vkc = (coords + base[k][None, None, :] * R)[:, :, None, :]
        ckm = jnp.clip(cck - vkc, -(R - r8) / 2, (R - r8) / 2) + vkc
        moves.append(ckm)
        dis_center = (truncate_xyz2 - ckm) / r8
        replace = jnp.all(jnp.abs(dis_center) <= 0.5, axis=-1) & (~mask_oob[:, :, k:k + 1])
        idx_scatter = jnp.where(replace, jnp.int32(k), idx_scatter)
    voxel_xyz = jnp.concatenate(moves, axis=2)
    ca = _scatter_add_last(truncated_corr, idx_scatter, num + 1)
    cnt = _scatter_add_last(jnp.ones_like(truncated_corr), idx_scatter, num + 1)
    voxel_corr = ca[:, :, :num] / jnp.clip(cnt[:, :, :num], 1.0, float(n_p))
    vin = jnp.concatenate(
        [voxel_corr[:, None], jnp.transpose(voxel_xyz, (0, 3, 1, 2))], axis=1)
    vfe = _conv2d(vin, W_v1, b_v1)
    vfe = _prelu(_group_norm(vfe, g_v, be_v, 8), a_v)
    vfe = _conv2d(vfe, W_v2, b_v2)
    corr_feature.append(jnp.transpose(vfe[:, 0], (0, 2, 1)))
    feat = jnp.concatenate(corr_feature, axis=1)
    h = _conv1d(feat, W_out1, b_out1)
    h = _prelu(_group_norm(h, g_out, be_out, 8), a_out)
    voxel_out = _conv1d(h, W_out2, b_out2)
    dist = jnp.sum(diff ** 2, axis=-1)
    _, neighbors = jax.lax.top_k(-dist, KNN)
    knn_corr = jnp.take_along_axis(truncated_corr, neighbors, axis=2)[:, None]
    knn_xyz = jnp.take_along_axis(
        truncate_xyz2,
        jnp.broadcast_to(neighbors[..., None], neighbors.shape + (3,)), axis=2)
    knn_xyz = jnp.transpose(knn_xyz, (0, 3, 1, 2)) - jnp.transpose(coords, (0, 2, 1))[..., None]
    kin = jnp.concatenate([knn_corr, knn_xyz], axis=1)
    kf = _conv2d(kin, W_k1, b_k1)
    kf = _prelu(_group_norm(kf, g_k, be_k, 8), a_k)
    kf = jnp.max(kf, axis=3)
    knn_feat = _conv1d(kf, W_ko, b_ko)
    return voxel_out + knn_feat


# speedup vs baseline: 6.3872x; 3.0907x over previous
import functools
import math

import jax
import jax.numpy as jnp
import numpy as np
from jax.experimental import pallas as pl

NUM_LEVELS = 3
BASE_SCALE = 0.25
RESOLUTION = 3
TRUNCATE_K = 128
KNN = 32


def _get_base():
    num = RESOLUTION ** 3
    base = np.zeros((num, 3), np.float32)
    for i in range(num):
        q = i
        for j in range(3):
            base[i][2 - j] = q % RESOLUTION - 1
            q = q // RESOLUTION
    return jnp.asarray(base)


def _conv1d(x, W, b):
    return jnp.einsum('oi,bin->bon', W, x) + b[None, :, None]


def _conv2d(x, W, b):
    return jnp.einsum('oi,bihw->bohw', W, x) + b[None, :, None, None]


def _group_norm(x, gamma, beta, g, eps=1e-5):
    b, C = x.shape[0], x.shape[1]
    rest = x.shape[2:]
    xr = x.reshape(b, g, -1)
    mean = xr.mean(axis=2, keepdims=True)
    var = xr.var(axis=2, keepdims=True)
    xn = ((xr - mean) / jnp.sqrt(var + eps)).reshape(x.shape)
    sh = (1, C) + (1,) * len(rest)
    return xn * gamma.reshape(sh) + beta.reshape(sh)


def _prelu(x, a):
    return jnp.where(x >= 0, x, a * x)


def _scatter_add_last(vals, idx, nbins):
    b, n, k = vals.shape
    bi = jnp.arange(b)[:, None, None]
    ni = jnp.arange(n)[None, :, None]
    return jnp.zeros((b, n, nbins), vals.dtype).at[bi, ni, idx].add(vals)


def _scatter_max_last(vals, idx, nbins):
    mask = idx[..., None] == jnp.arange(nbins)[None, None, None, :]
    masked = jnp.where(mask, vals[..., None], -jnp.inf)
    return masked.max(axis=2), masked.argmax(axis=2)


_MIN32 = -2147483648


def _corr_topk_body(f1_ref, f2_ref, xyz_ref,
                    tc_ref, dx_ref, dy_ref, dz_ref, *, bn, n, k):
    i32, f32 = jnp.int32, jnp.float32
    nck = n // 128
    f1 = f1_ref[0]
    f2 = f2_ref[0]
    corr = jax.lax.dot_general(
        f1, f2, (((0,), (0,)), ((), ())), preferred_element_type=f32)

    corr3 = corr.reshape(bn, nck, 128)
    kbits = jax.lax.bitcast_convert_type(corr3, i32)
    ks3 = kbits ^ ((kbits >> 31) & jnp.int32(0x7FFFFFFF))

    kf = jnp.float32(k)

    def _bisect(i, tb):
        bit = jnp.left_shift(jnp.int32(1), 31 - i)
        cand = tb | bit
        cand_s = (cand ^ jnp.int32(_MIN32))[:, :, None]
        cnt = jnp.sum(jnp.sum((ks3 >= cand_s).astype(f32), axis=2),
                      axis=1, keepdims=True)
        return jnp.where(cnt >= kf, cand, tb)

    tb = jax.lax.fori_loop(0, 32, _bisect, jnp.zeros((bn, 1), i32))
    ts3 = (tb ^ jnp.int32(_MIN32))[:, :, None]

    gt3 = ks3 > ts3
    eq3b = ks3 == ts3
    cnt_gt = jnp.sum(jnp.sum(gt3.astype(f32), axis=2), axis=1, keepdims=True)
    m_tie = kf - cnt_gt

    ir = jax.lax.broadcasted_iota(i32, (128, 128), 0)
    ic = jax.lax.broadcasted_iota(i32, (128, 128), 1)
    tri = (ir <= ic).astype(f32)
    ir2 = jax.lax.broadcasted_iota(i32, (nck, nck), 0)
    ic2 = jax.lax.broadcasted_iota(i32, (nck, nck), 1)
    tri_c_incl = (ir2 <= ic2).astype(f32)

    eq3 = eq3b.astype(f32)
    wc_eq = jax.lax.dot_general(eq3, tri, (((2,), (0,)), ((), ())),
                                preferred_element_type=f32)
    tot_eq = jnp.sum(eq3, axis=2)
    oi_eq = jax.lax.dot_general(tot_eq, tri_c_incl, (((1,), (0,)), ((), ())),
                                preferred_element_type=f32)
    off_eq = oi_eq - tot_eq
    rank_excl = off_eq[:, :, None] + (wc_eq - eq3)
    sel = gt3 | (eq3b & (rank_excl < m_tie[:, :, None]))

    sf = sel.astype(f32)
    cw = jax.lax.dot_general(sf, tri, (((2,), (0,)), ((), ())),
                             preferred_element_type=f32)
    m_c = jnp.sum(sf, axis=2)
    oi = jax.lax.dot_general(m_c, tri_c_incl, (((1,), (0,)), ((), ())),
                             preferred_element_type=f32)
    se = oi - m_c

    kio_f = jax.lax.broadcasted_iota(i32, (bn, k), 1).astype(f32)
    cstar = jnp.zeros((bn, k), i32)
    for c in range(nck):
        cstar = cstar + (oi[:, c:c + 1] <= kio_f).astype(i32)
    se_k = jnp.take_along_axis(se, cstar, axis=1)
    s_loc = kio_f - se_k

    cwi = cw.astype(i32)
    sio = jax.lax.broadcasted_iota(i32, (bn, nck, 128), 2)

    def _ssorted(i, q3):
        b_ = jnp.int32(64) >> i
        cand = q3 + b_
        cc = jnp.take_along_axis(cwi, cand - 1, axis=2)
        return jnp.where(cc <= sio, cand, q3)

    q3 = jax.lax.fori_loop(0, 7, _ssorted, jnp.zeros((bn, nck, 128), i32))

    s_loc_i = s_loc.astype(i32)
    out_v = jnp.zeros((bn, k), f32)
    out_x = jnp.zeros((bn, k), f32)
    out_y = jnp.zeros((bn, k), f32)
    out_z = jnp.zeros((bn, k), f32)
    xyz = xyz_ref[0]
    for c in range(nck):
        msk = cstar == c
        qv = jnp.take_along_axis(q3[:, c, :], s_loc_i, axis=1)
        vv = jnp.take_along_axis(corr3[:, c, :], qv, axis=1)
        xc = jnp.broadcast_to(xyz[0:1, c * 128:(c + 1) * 128], (bn, k))
        yc = jnp.broadcast_to(xyz[1:2, c * 128:(c + 1) * 128], (bn, k))
        zc = jnp.broadcast_to(xyz[2:3, c * 128:(c + 1) * 128], (bn, k))
        out_v = jnp.where(msk, vv, out_v)
        out_x = jnp.where(msk, jnp.take_along_axis(xc, qv, axis=1), out_x)
        out_y = jnp.where(msk, jnp.take_along_axis(yc, qv, axis=1), out_y)
        out_z = jnp.where(msk, jnp.take_along_axis(zc, qv, axis=1), out_z)

    am = jnp.argmax(out_v, axis=1, keepdims=True).astype(i32)
    lane = jax.lax.broadcasted_iota(i32, (bn, k), 1)

    def _swap0(p):
        pm = jnp.take_along_axis(p, am, axis=1)
        p0 = p[:, 0:1]
        return jnp.where(lane == 0, pm, jnp.where(lane == am, p0, p))

    out_v, out_x, out_y, out_z = map(_swap0, (out_v, out_x, out_y, out_z))

    scale = jnp.sqrt(jnp.float32(f1.shape[0]))
    tc_ref[0] = out_v / scale
    dx_ref[0] = out_x
    dy_ref[0] = out_y
    dz_ref[0] = out_z


def _corr_topk(fmap1, fmap2, xyz2):
    b, d, n = fmap1.shape
    k = TRUNCATE_K
    BN = 256
    xyz8 = jnp.concatenate(
        [jnp.transpose(xyz2, (0, 2, 1)),
         jnp.zeros((b, 5, n), jnp.float32)], axis=1)
    grid = (b, n // BN)
    out = pl.pallas_call(
        functools.partial(_corr_topk_body, bn=BN, n=n, k=k),
        grid=grid,
        in_specs=[
            pl.BlockSpec((1, d, BN), lambda i, j: (i, 0, j)),
            pl.BlockSpec((1, d, n), lambda i, j: (i, 0, 0)),
            pl.BlockSpec((1, 8, n), lambda i, j: (i, 0, 0)),
        ],
        out_specs=[pl.BlockSpec((1, BN, k), lambda i, j: (i, j, 0))] * 4,
        out_shape=[jax.ShapeDtypeStruct((b, n, k), jnp.float32)] * 4,
    )(fmap1, fmap2, xyz8)
    return out


def _voxel_knn_body(tc_ref, tx_ref, ty_ref, tz_ref, co_ref, wk_ref, bk_ref,
                    cf_ref, vox_ref, km_ref, ks_ref, *, bn, k):
    i32, f32 = jnp.int32, jnp.float32
    NEG = jnp.float32(-jnp.inf)
    tc = tc_ref[0]
    tx = tx_ref[0]
    ty = ty_ref[0]
    tz = tz_ref[0]
    cx = co_ref[0, :, 0:1]
    cy = co_ref[0, :, 1:2]
    cz = co_ref[0, :, 2:3]
    dgx = tx - cx
    dgy = ty - cy
    dgz = tz - cz
    npf = jnp.float32(4096.0)

    for lev in range(3):
        r = jnp.float32(BASE_SCALE * 2 ** lev)
        dvx = jnp.round(dgx / r)
        dvy = jnp.round(dgy / r)
        dvz = jnp.round(dgz / r)
        valid = ((jnp.abs(dvx) <= 1.0) & (jnp.abs(dvy) <= 1.0)
                 & (jnp.abs(dvz) <= 1.0))
        cube = (dvx + 1.0) * 9.0 + (dvy + 1.0) * 3.0 + (dvz + 1.0)
        cube_i = jnp.where(valid, cube, 0.0).astype(i32)
        vf = valid.astype(f32)
        tv = tc * vf
        for b_ in range(27):
            m = ((cube_i == b_).astype(f32)) * vf
            ca = jnp.sum(tv * m, axis=1, keepdims=True)
            cc = jnp.clip(jnp.sum(m, axis=1, keepdims=True), 1.0, npf)
            cf_ref[0, :, lev * 27 + b_:lev * 27 + b_ + 1] = ca / cc

    dvxc = jnp.round(dgx / 4.0)
    dvyc = jnp.round(dgy / 4.0)
    dvzc = jnp.round(dgz / 4.0)
    validc = ((jnp.abs(dvxc) <= 1.5) & (jnp.abs(dvyc) <= 1.5)
              & (jnp.abs(dvzc) <= 1.5))
    cidx = (dvxc + 1.0) * 9.0 + (dvyc + 1.0) * 3.0 + (dvzc + 1.0)
    full = jnp.where(validc, cidx + 1.0, 0.0).astype(i32)
    mvals = tc * validc.astype(f32)
    idx_sc = jnp.full((bn, k), 27, i32)
    for kk in range(27):
        m = full == (kk + 1)
        mv = jnp.where(m, mvals, NEG)
        ccb = jnp.max(mv, axis=1, keepdims=True)
        cib = jnp.argmax(mv, axis=1, keepdims=True).astype(i32)
        oob = ccb <= 0.0
        cib = jnp.where(oob, 0, cib)
        ccx = jnp.take_along_axis(tx, cib, axis=1)
        ccy = jnp.take_along_axis(ty, cib, axis=1)
        ccz = jnp.take_along_axis(tz, cib, axis=1)
        bx = jnp.float32(kk // 9 - 1)
        by = jnp.float32((kk // 3) % 3 - 1)
        bz = jnp.float32(kk % 3 - 1)
        vkx = cx + bx * 4.0
        vky = cy + by * 4.0
        vkz = cz + bz * 4.0
        ckmx = jnp.clip(ccx - vkx, -1.0, 1.0) + vkx
        ckmy = jnp.clip(ccy - vky, -1.0, 1.0) + vky
        ckmz = jnp.clip(ccz - vkz, -1.0, 1.0) + vkz
        vox_ref[0, :, 27 + kk:27 + kk + 1] = ckmx
        vox_ref[0, :, 54 + kk:54 + kk + 1] = ckmy
        vox_ref[0, :, 81 + kk:81 + kk + 1] = ckmz
        rep = ((jnp.abs((tx - ckmx) / 2.0) <= 0.5)
               & (jnp.abs((ty - ckmy) / 2.0) <= 0.5)
               & (jnp.abs((tz - ckmz) / 2.0) <= 0.5)
               & jnp.logical_not(oob))
        idx_sc = jnp.where(rep, jnp.int32(kk), idx_sc)

    for kk in range(27):
        m = (idx_sc == kk).astype(f32)
        ca = jnp.sum(tc * m, axis=1, keepdims=True)
        cnt = jnp.clip(jnp.sum(m, axis=1, keepdims=True), 1.0, npf)
        vox_ref[0, :, kk:kk + 1] = ca / cnt

    dist = dgx * dgx + dgy * dgy + dgz * dgz
    kb = jax.lax.bitcast_convert_type(dist, i32)
    ksn = -kb
    kf32 = jnp.float32(KNN)

    def _nbis(i, tb):
        bit = jnp.left_shift(jnp.int32(1), 31 - i)
        cand = tb | bit
        cand_s = cand ^ jnp.int32(_MIN32)
        cnt = jnp.sum((ksn >= cand_s).astype(f32), axis=1, keepdims=True)
        return jnp.where(cnt >= kf32, cand, tb)

    tbn = jax.lax.fori_loop(0, 32, _nbis, jnp.zeros((bn, 1), i32))
    tsn = tbn ^ jnp.int32(_MIN32)
    gtn = ksn > tsn
    eqn = ksn == tsn
    eqf = eqn.astype(f32)
    cntg = jnp.sum(gtn.astype(f32), axis=1, keepdims=True)
    mt = kf32 - cntg
    ir = jax.lax.broadcasted_iota(i32, (k, k), 0)
    ic = jax.lax.broadcasted_iota(i32, (k, k), 1)
    tri = (ir <= ic).astype(f32)
    wcn = jax.lax.dot_general(eqf, tri, (((1,), (0,)), ((), ())),
                              preferred_element_type=f32)
    seln = gtn | (eqn & ((wcn - eqf) < mt))

    wk = wk_ref[...]
    bk = bk_ref[...]
    for co in range(64):
        wq = (wk[co, 0] * tc + wk[co, 1] * dgx + wk[co, 2] * dgy
              + wk[co, 3] * dgz + bk[0, co])
        qm = jnp.where(seln, wq, NEG)
        km_ref[0, :, co:co + 1] = jnp.max(qm, axis=1, keepdims=True)
        qs = jnp.where(seln, wq, 0.0)
        ks_ref[0, :, co:co + 1] = jnp.sum(qs, axis=1, keepdims=True)
        ks_ref[0, :, 64 + co:64 + co + 1] = jnp.sum(qs * qs, axis=1,
                                                    keepdims=True)


def _voxel_knn(tc, tx, ty, tz, coords, W_k1, b_k1):
    b, n, k = tc.shape
    BN = 128
    grid = (b, n // BN)
    bspec = pl.BlockSpec((1, BN, k), lambda i, j: (i, j, 0))
    out = pl.pallas_call(
        functools.partial(_voxel_knn_body, bn=BN, k=k),
        grid=grid,
        in_specs=[
            bspec, bspec, bspec, bspec,
            pl.BlockSpec((1, BN, 3), lambda i, j: (i, j, 0)),
            pl.BlockSpec((64, 4), lambda i, j: (0, 0)),
            pl.BlockSpec((1, 64), lambda i, j: (0, 0)),
        ],
        out_specs=[
            pl.BlockSpec((1, BN, 81), lambda i, j: (i, j, 0)),
            pl.BlockSpec((1, BN, 108), lambda i, j: (i, j, 0)),
            pl.BlockSpec((1, BN, 64), lambda i, j: (i, j, 0)),
            pl.BlockSpec((1, BN, 128), lambda i, j: (i, j, 0)),
        ],
        out_shape=[
            jax.ShapeDtypeStruct((b, n, 81), jnp.float32),
            jax.ShapeDtypeStruct((b, n, 108), jnp.float32),
            jax.ShapeDtypeStruct((b, n, 64), jnp.float32),
            jax.ShapeDtypeStruct((b, n, 128), jnp.float32),
        ],
    )(tc, tx, ty, tz, coords, W_k1, b_k1.reshape(1, 64))
    return out


def kernel(fmap1, fmap2, xyz2, coords, W_out1, b_out1, g_out, be_out, a_out,
           W_out2, b_out2, W_v1, b_v1, g_v, be_v, a_v, W_v2, b_v2,
           W_k1, b_k1, g_k, be_k, a_k, W_ko, b_ko):
    b, dim, n_p = fmap1.shape
    K = TRUNCATE_K
    num = RESOLUTION ** 3
    truncated_corr, ttx, tty, ttz = _corr_topk(fmap1, fmap2, xyz2)
    cf, vox, kmax, kstat = _voxel_knn(truncated_corr, ttx, tty, ttz,
                                      coords, W_k1, b_k1)

    voxel_corr = vox[:, :, :num]
    vin = jnp.stack([voxel_corr, vox[:, :, 27:54], vox[:, :, 54:81],
                     vox[:, :, 81:108]], axis=1)
    vfe = _conv2d(vin, W_v1, b_v1)
    vfe = _prelu(_group_norm(vfe, g_v, be_v, 8), a_v)
    vfe = _conv2d(vfe, W_v2, b_v2)
    feat = jnp.concatenate([jnp.transpose(cf, (0, 2, 1)),
                            jnp.transpose(vfe[:, 0], (0, 2, 1))], axis=1)
    h = _conv1d(feat, W_out1, b_out1)
    h = _prelu(_group_norm(h, g_out, be_out, 8), a_out)
    voxel_out = _conv1d(h, W_out2, b_out2)

    s1 = jnp.sum(kstat[:, :, :64], axis=1)
    s2 = jnp.sum(kstat[:, :, 64:], axis=1)
    g1 = jnp.sum(s1.reshape(b, 8, 8), axis=2)
    g2 = jnp.sum(s2.reshape(b, 8, 8), axis=2)
    cnt = jnp.float32(8 * n_p * KNN)
    mu = g1 / cnt
    var = g2 / cnt - mu * mu
    rstd = 1.0 / jnp.sqrt(var + 1e-5)
    kmax_t = jnp.transpose(kmax, (0, 2, 1))
    mu_c = jnp.repeat(mu, 8, axis=1)[:, :, None]
    rstd_c = jnp.repeat(rstd, 8, axis=1)[:, :, None]
    kf = (kmax_t - mu_c) * rstd_c * g_k[None, :, None] + be_k[None, :, None]
    kf = _prelu(kf, a_k)
    knn_feat = _conv1d(kf, W_ko, b_ko)
    return voxel_out + knn_feat
